# baseline (speedup 1.0000x reference)
import functools

import jax
import jax.numpy as jnp
from jax.experimental import pallas as pl


_NBLK = 2000
_EBLK = 2000


def _mm_body(x_ref, w_ref, b_ref, o_ref, *, act):
    y = jnp.dot(x_ref[...], w_ref[...], preferred_element_type=jnp.float32)
    y = y + b_ref[...]
    if act:
        y = jnp.maximum(y, 0.0)
    o_ref[...] = y


def _mm(x, w, b, act=True, blk=_NBLK):
    n, d = x.shape
    dout = w.shape[1]
    if n % blk != 0:
        blk = 8
        pad = (-n) % blk
        x = jnp.pad(x, ((0, pad), (0, 0)))
    np_ = x.shape[0]
    out = pl.pallas_call(
        functools.partial(_mm_body, act=act),
        grid=(np_ // blk,),
        in_specs=[
            pl.BlockSpec((blk, d), lambda i: (i, 0)),
            pl.BlockSpec((d, dout), lambda i: (0, 0)),
            pl.BlockSpec((1, dout), lambda i: (0, 0)),
        ],
        out_specs=pl.BlockSpec((blk, dout), lambda i: (i, 0)),
        out_shape=jax.ShapeDtypeStruct((np_, dout), jnp.float32),
    )(x, w, b.reshape(1, -1))
    return out[:n]


def _edge_mlp_e_body(xi_ref, xj_ref, ef_ref, wa_ref, wb_ref, wc_ref, b1_ref,
                     w2_ref, b2_ref, o_ref):
    xi = xi_ref[...]
    t = xj_ref[...] - xi
    h = jnp.dot(xi, wa_ref[...], preferred_element_type=jnp.float32)
    h = h + jnp.dot(t, wb_ref[...], preferred_element_type=jnp.float32)
    h = h + jnp.dot(ef_ref[...], wc_ref[...], preferred_element_type=jnp.float32)
    h = jnp.maximum(h + b1_ref[...], 0.0)
    y = jnp.dot(h, w2_ref[...], preferred_element_type=jnp.float32) + b2_ref[...]
    o_ref[...] = jnp.maximum(y, 0.0)


def _edge_mlp_body(xi_ref, xj_ref, wa_ref, wb_ref, b1_ref, w2_ref, b2_ref, o_ref):
    xi = xi_ref[...]
    t = xj_ref[...] - xi
    h = jnp.dot(xi, wa_ref[...], preferred_element_type=jnp.float32)
    h = h + jnp.dot(t, wb_ref[...], preferred_element_type=jnp.float32)
    h = jnp.maximum(h + b1_ref[...], 0.0)
    y = jnp.dot(h, w2_ref[...], preferred_element_type=jnp.float32) + b2_ref[...]
    o_ref[...] = jnp.maximum(y, 0.0)


def _edge_mlp(xi, xj, efeat, p):
    e, d = xi.shape
    w1 = p["l1"]["W"]
    b1 = p["l1"]["b"]
    w2 = p["l2"]["W"]
    b2 = p["l2"]["b"]
    dh = w1.shape[1]
    dout = w2.shape[1]
    wa = w1[:d]
    wb = w1[d : 2 * d]
    blk = _EBLK
    if efeat is not None:
        de = efeat.shape[1]
        wc = w1[2 * d :]
        return pl.pallas_call(
            _edge_mlp_e_body,
            grid=(e // blk,),
            in_specs=[
                pl.BlockSpec((blk, d), lambda i: (i, 0)),
                pl.BlockSpec((blk, d), lambda i: (i, 0)),
                pl.BlockSpec((blk, de), lambda i: (i, 0)),
                pl.BlockSpec((d, dh), lambda i: (0, 0)),
                pl.BlockSpec((d, dh), lambda i: (0, 0)),
                pl.BlockSpec((de, dh), lambda i: (0, 0)),
                pl.BlockSpec((1, dh), lambda i: (0, 0)),
                pl.BlockSpec((dh, dout), lambda i: (0, 0)),
                pl.BlockSpec((1, dout), lambda i: (0, 0)),
            ],
            out_specs=pl.BlockSpec((blk, dout), lambda i: (i, 0)),
            out_shape=jax.ShapeDtypeStruct((e, dout), jnp.float32),
        )(xi, xj, efeat, wa, wb, wc, b1.reshape(1, -1), w2, b2.reshape(1, -1))
    return pl.pallas_call(
        _edge_mlp_body,
        grid=(e // blk,),
        in_specs=[
            pl.BlockSpec((blk, d), lambda i: (i, 0)),
            pl.BlockSpec((blk, d), lambda i: (i, 0)),
            pl.BlockSpec((d, dh), lambda i: (0, 0)),
            pl.BlockSpec((d, dh), lambda i: (0, 0)),
            pl.BlockSpec((1, dh), lambda i: (0, 0)),
            pl.BlockSpec((dh, dout), lambda i: (0, 0)),
            pl.BlockSpec((1, dout), lambda i: (0, 0)),
        ],
        out_specs=pl.BlockSpec((blk, dout), lambda i: (i, 0)),
        out_shape=jax.ShapeDtypeStruct((e, dout), jnp.float32),
    )(xi, xj, wa, wb, b1.reshape(1, -1), w2, b2.reshape(1, -1))


def _conv(x, src, dst, p, efeat=None):
    xi = jnp.take(x, dst, axis=0)
    xj = jnp.take(x, src, axis=0)
    msg = _edge_mlp(xi, xj, efeat, p)
    agg = jax.ops.segment_max(msg, dst, num_segments=x.shape[0])
    return jnp.maximum(agg, 0.0)


def kernel(coordinates, adjacency, node_features, edge_features, joint_types, params):
    src = adjacency[0]
    dst = adjacency[1]
    x = _mm(coordinates, params["hid1"]["W"], params["hid1"]["b"])
    x = _mm(x, params["hid2"]["W"], params["hid2"]["b"])
    x = jnp.concatenate([x, node_features, joint_types], axis=-1)
    x = _conv(x, src, dst, params["ece1"], edge_features)
    x = _conv(x, src, dst, params["ece2"], edge_features)
    x = _mm(x, params["hid3"]["W"], params["hid3"]["b"])
    x = _conv(x, src, dst, params["ec1"])
    ec1_out = x
    x = _conv(x, src, dst, params["ec2"])
    ec2_out = x
    x = _conv(jnp.concatenate([x, ec1_out], axis=-1), src, dst, params["ec3"])
    x2 = jnp.concatenate([x, ec2_out], axis=-1)
    x1 = _mm(x2, params["hid4"]["W"], params["hid4"]["b"])
    x1 = _mm(x1, params["out1"]["W"], params["out1"]["b"])
    x2 = _mm(x2, params["hid5"]["W"], params["hid5"]["b"])
    x2 = _mm(x2, params["out2"]["W"], params["out2"]["b"])
    return (x1, x2)
